# Initial kernel scaffold; baseline (speedup 1.0000x reference)
import jax
import jax.numpy as jnp
from jax.experimental import pallas as pl


def kernel(x, edge_index, W1, b1, W2, b2):
    raise NotImplementedError("write your pallas kernel here")


# speedup vs baseline: 2.8270x; 2.8270x over previous
import functools

import jax
import jax.numpy as jnp
from jax import lax
from jax.experimental import pallas as pl
from jax.experimental.pallas import tpu as pltpu
from jax.experimental.pallas import tpu_sc as plsc

N = 10000
E = 320000
D = 128

NP = 10240
TRASH = N
L = 128
CH_AGG = 80
CH_HIST = 160
EPAD = 32 * CH_AGG * L
ROWS_PER_TILE = NP // 16


def _mesh():
    return plsc.VectorSubcoreMesh(core_axis_name="c", subcore_axis_name="s")


def _sc_hist_body(srcdst, zeros128, ones128, out, idx_v, ones_v, acc, sem):
    c = lax.axis_index("c")
    s = lax.axis_index("s")
    base = s * ROWS_PER_TILE

    pltpu.sync_copy(zeros128, ones_v)
    for j in range(ROWS_PER_TILE // L):
        pltpu.sync_copy(ones_v, acc.at[pl.ds(base + j * L, L)])
    pltpu.sync_copy(ones128, ones_v)
    pltpu.sync_copy(srcdst.at[c, pl.ds(s * CH_HIST, CH_HIST)], idx_v)
    plsc.subcore_barrier()

    def body(j, _):
        pltpu.sync_copy(ones_v, acc.at[idx_v.at[j]], add=True)
        return ()

    lax.fori_loop(0, CH_HIST, body, ())
    plsc.subcore_barrier()

    for j in range(ROWS_PER_TILE // L):
        pltpu.sync_copy(acc.at[pl.ds(base + j * L, L)], ones_v)
        pltpu.sync_copy(ones_v, out.at[c, pl.ds(base + j * L, L)])


def _sc_hist(srcdst, zeros128, ones128):
    return pl.kernel(
        _sc_hist_body,
        out_type=jax.ShapeDtypeStruct((2, NP, D), jnp.float32),
        mesh=_mesh(),
        scratch_types=[
            pltpu.VMEM((CH_HIST, L), jnp.int32),
            pltpu.VMEM((L, D), jnp.float32),
            pltpu.VMEM_SHARED((NP, D), jnp.float32),
            pltpu.SemaphoreType.DMA,
        ],
    )(srcdst, zeros128, ones128)


def _sc_agg_body(table, e3d, zeros128, out, ib0, rows0, acc, sem0):
    c = lax.axis_index("c")
    s = lax.axis_index("s")
    b = (c * 16 + s) * CH_AGG
    base = s * ROWS_PER_TILE

    pltpu.sync_copy(zeros128, rows0)
    for j in range(ROWS_PER_TILE // L):
        pltpu.sync_copy(rows0, acc.at[pl.ds(base + j * L, L)])
    plsc.subcore_barrier()

    def body(g, _):
        pltpu.sync_copy(e3d.at[b + g], ib0)
        pltpu.async_copy(table.at[ib0.at[0]], rows0, sem0).wait()
        pltpu.sync_copy(rows0, acc.at[ib0.at[1]], add=True)
        return ()

    lax.fori_loop(0, CH_AGG, body, ())
    plsc.subcore_barrier()

    for j in range(ROWS_PER_TILE // L):
        pltpu.sync_copy(acc.at[pl.ds(base + j * L, L)], rows0)
        pltpu.sync_copy(rows0, out.at[c, pl.ds(base + j * L, L)])


def _sc_agg(table, e3d, zeros128):
    return pl.kernel(
        _sc_agg_body,
        out_type=jax.ShapeDtypeStruct((2, NP, D), jnp.float32),
        mesh=_mesh(),
        scratch_types=[
            pltpu.VMEM((2, L), jnp.int32),
            pltpu.VMEM((L, D), jnp.float32),
            pltpu.VMEM_SHARED((NP, D), jnp.float32),
            pltpu.SemaphoreType.DMA,
        ],
    )(table, e3d, zeros128)


def _norm(deg_col):
    return lax.rsqrt(jnp.where(deg_col > 0.0, deg_col, 1.0))


def _tc_scale_body(x_ref, deg_ref, o_ref):
    o_ref[...] = x_ref[...] * _norm(deg_ref[...])


def _tc_scale(x_pad, deg_out_col):
    blk = 2048
    return pl.pallas_call(
        _tc_scale_body,
        out_shape=jax.ShapeDtypeStruct((NP, D), jnp.float32),
        grid=(NP // blk,),
        in_specs=[
            pl.BlockSpec((blk, D), lambda i: (i, 0)),
            pl.BlockSpec((blk, 1), lambda i: (i, 0)),
        ],
        out_specs=pl.BlockSpec((blk, D), lambda i: (i, 0)),
    )(x_pad, deg_out_col)


def _tc_stage_body(relu_scale, p_ref, din_ref, dout_ref, w_ref, b_ref, o_ref):
    m = (p_ref[0] + p_ref[1]) * _norm(din_ref[...])
    h = jnp.dot(m, w_ref[...], preferred_element_type=jnp.float32)
    h = h + b_ref[...]
    if relu_scale:
        h = jnp.maximum(h, 0.0) * _norm(dout_ref[...])
    o_ref[...] = h


def _tc_stage(p, deg_in_col, deg_out_col, w, b, relu_scale):
    blk = 2048
    return pl.pallas_call(
        functools.partial(_tc_stage_body, relu_scale),
        out_shape=jax.ShapeDtypeStruct((NP, D), jnp.float32),
        grid=(NP // blk,),
        in_specs=[
            pl.BlockSpec((2, blk, D), lambda i: (0, i, 0)),
            pl.BlockSpec((blk, 1), lambda i: (i, 0)),
            pl.BlockSpec((blk, 1), lambda i: (i, 0)),
            pl.BlockSpec((D, D), lambda i: (0, 0)),
            pl.BlockSpec((1, D), lambda i: (0, 0)),
        ],
        out_specs=pl.BlockSpec((blk, D), lambda i: (i, 0)),
    )(p, deg_in_col, deg_out_col, w, b)


def kernel(x, edge_index, W1, b1, W2, b2):
    pad = jnp.full((EPAD - E,), TRASH, dtype=jnp.int32)
    src2d = jnp.concatenate([edge_index[0], pad]).reshape(EPAD // L, L)
    dst2d = jnp.concatenate([edge_index[1], pad]).reshape(EPAD // L, L)
    srcdst = jnp.stack([src2d, dst2d])
    e3d = jnp.stack([src2d, dst2d], axis=1)

    zeros128 = jnp.zeros((L, D), jnp.float32)
    ones128 = jnp.ones((L, D), jnp.float32)
    x_pad = jnp.pad(x, ((0, NP - N), (0, 0)))

    hist = _sc_hist(srcdst, zeros128, ones128)
    deg_out_col = hist[0, :, 0:1]
    deg_in_col = hist[1, :, 0:1]

    t1 = _tc_scale(x_pad, deg_out_col)
    p1 = _sc_agg(t1, e3d, zeros128)
    t2 = _tc_stage(p1, deg_in_col, deg_out_col, W1, b1.reshape(1, D),
                   relu_scale=True)
    p2 = _sc_agg(t2, e3d, zeros128)
    out = _tc_stage(p2, deg_in_col, deg_out_col, W2, b2.reshape(1, D),
                    relu_scale=False)
    return out[:N]


# speedup vs baseline: 8.1715x; 2.8905x over previous
import functools

import jax
import jax.numpy as jnp
from jax import lax
from jax.experimental import pallas as pl
from jax.experimental.pallas import tpu as pltpu
from jax.experimental.pallas import tpu_sc as plsc

N = 10000
E = 320000
D = 128

NP = 10240
TRASH = N
L = 128
CH_AGG = 80
CH_HIST = 160
EPAD = 32 * CH_AGG * L
ROWS_PER_TILE = NP // 16


def _mesh():
    return plsc.VectorSubcoreMesh(core_axis_name="c", subcore_axis_name="s")


def _sc_hist_body(srcdst, zeros128, ones128, out, idx_v, ones_v, acc, sem):
    c = lax.axis_index("c")
    s = lax.axis_index("s")
    base = s * ROWS_PER_TILE

    pltpu.sync_copy(zeros128, ones_v)
    for j in range(ROWS_PER_TILE // L):
        pltpu.sync_copy(ones_v, acc.at[pl.ds(base + j * L, L)])
    pltpu.sync_copy(ones128, ones_v)
    pltpu.sync_copy(srcdst.at[c, pl.ds(s * CH_HIST, CH_HIST)], idx_v)
    plsc.subcore_barrier()

    K = 8

    def body(g, _):
        for k in range(K):
            pltpu.make_async_copy(
                ones_v, acc.at[idx_v.at[g * K + k]], sem).start(add=True)
        for k in range(K):
            pltpu.make_async_copy(ones_v, acc.at[idx_v.at[0]], sem).wait()
        return ()

    lax.fori_loop(0, CH_HIST // K, body, ())
    plsc.subcore_barrier()

    for j in range(ROWS_PER_TILE // L):
        pltpu.sync_copy(acc.at[pl.ds(base + j * L, L)], ones_v)
        pltpu.sync_copy(ones_v, out.at[c, pl.ds(base + j * L, L)])


def _sc_hist(srcdst, zeros128, ones128):
    return pl.kernel(
        _sc_hist_body,
        out_type=jax.ShapeDtypeStruct((2, NP, D), jnp.float32),
        mesh=_mesh(),
        scratch_types=[
            pltpu.VMEM((CH_HIST, L), jnp.int32),
            pltpu.VMEM((L, D), jnp.float32),
            pltpu.VMEM_SHARED((NP, D), jnp.float32),
            pltpu.SemaphoreType.DMA,
        ],
    )(srcdst, zeros128, ones128)


def _sc_agg_body(table, e3d, zeros128, out, ibA, ibB, rowsA, rowsB, acc,
                 semA, semB):
    c = lax.axis_index("c")
    s = lax.axis_index("s")
    b = (c * 16 + s) * CH_AGG
    base = s * ROWS_PER_TILE

    pltpu.sync_copy(zeros128, rowsA)
    for j in range(ROWS_PER_TILE // L):
        pltpu.sync_copy(rowsA, acc.at[pl.ds(base + j * L, L)])
    plsc.subcore_barrier()

    pltpu.sync_copy(e3d.at[b], ibA)
    pltpu.sync_copy(e3d.at[b + 1], ibB)
    pltpu.async_copy(table.at[ibA.at[0]], rowsA, semA)
    pltpu.async_copy(table.at[ibB.at[0]], rowsB, semB)

    def half(g, ib, rows, sem):
        pltpu.make_async_copy(table.at[ib.at[0]], rows, sem).wait()
        pltpu.sync_copy(rows, acc.at[ib.at[1]], add=True)
        pltpu.sync_copy(e3d.at[g + 2], ib)
        pltpu.async_copy(table.at[ib.at[0]], rows, sem)

    def body(g, _):
        half(b + 2 * g, ibA, rowsA, semA)
        half(b + 2 * g + 1, ibB, rowsB, semB)
        return ()

    lax.fori_loop(0, CH_AGG // 2 - 1, body, ())
    pltpu.make_async_copy(table.at[ibA.at[0]], rowsA, semA).wait()
    pltpu.sync_copy(rowsA, acc.at[ibA.at[1]], add=True)
    pltpu.make_async_copy(table.at[ibB.at[0]], rowsB, semB).wait()
    pltpu.sync_copy(rowsB, acc.at[ibB.at[1]], add=True)
    plsc.subcore_barrier()

    for j in range(ROWS_PER_TILE // L):
        pltpu.sync_copy(acc.at[pl.ds(base + j * L, L)], rowsA)
        pltpu.sync_copy(rowsA, out.at[c, pl.ds(base + j * L, L)])


def _sc_agg(table, e3d, zeros128):
    return pl.kernel(
        _sc_agg_body,
        out_type=jax.ShapeDtypeStruct((2, NP, D), jnp.float32),
        mesh=_mesh(),
        scratch_types=[
            pltpu.VMEM((2, L), jnp.int32),
            pltpu.VMEM((2, L), jnp.int32),
            pltpu.VMEM((L, D), jnp.float32),
            pltpu.VMEM((L, D), jnp.float32),
            pltpu.VMEM_SHARED((NP, D), jnp.float32),
            pltpu.SemaphoreType.DMA,
            pltpu.SemaphoreType.DMA,
        ],
    )(table, e3d, zeros128)


def _norm(deg_col):
    return lax.rsqrt(jnp.where(deg_col > 0.0, deg_col, 1.0))


def _tc_scale_body(x_ref, deg_ref, o_ref):
    o_ref[...] = x_ref[...] * _norm(deg_ref[...])


def _tc_scale(x_pad, deg_out_col):
    blk = 2048
    return pl.pallas_call(
        _tc_scale_body,
        out_shape=jax.ShapeDtypeStruct((NP, D), jnp.float32),
        grid=(NP // blk,),
        in_specs=[
            pl.BlockSpec((blk, D), lambda i: (i, 0)),
            pl.BlockSpec((blk, 1), lambda i: (i, 0)),
        ],
        out_specs=pl.BlockSpec((blk, D), lambda i: (i, 0)),
    )(x_pad, deg_out_col)


def _tc_stage_body(relu_scale, p_ref, din_ref, dout_ref, w_ref, b_ref, o_ref):
    m = (p_ref[0] + p_ref[1]) * _norm(din_ref[...])
    h = jnp.dot(m, w_ref[...], preferred_element_type=jnp.float32)
    h = h + b_ref[...]
    if relu_scale:
        h = jnp.maximum(h, 0.0) * _norm(dout_ref[...])
    o_ref[...] = h


def _tc_stage(p, deg_in_col, deg_out_col, w, b, relu_scale):
    blk = 2048
    return pl.pallas_call(
        functools.partial(_tc_stage_body, relu_scale),
        out_shape=jax.ShapeDtypeStruct((NP, D), jnp.float32),
        grid=(NP // blk,),
        in_specs=[
            pl.BlockSpec((2, blk, D), lambda i: (0, i, 0)),
            pl.BlockSpec((blk, 1), lambda i: (i, 0)),
            pl.BlockSpec((blk, 1), lambda i: (i, 0)),
            pl.BlockSpec((D, D), lambda i: (0, 0)),
            pl.BlockSpec((1, D), lambda i: (0, 0)),
        ],
        out_specs=pl.BlockSpec((blk, D), lambda i: (i, 0)),
    )(p, deg_in_col, deg_out_col, w, b)


def kernel(x, edge_index, W1, b1, W2, b2):
    pad = jnp.arange(EPAD - E, dtype=jnp.int32) % (NP - N) + TRASH
    src2d = jnp.concatenate([edge_index[0], pad]).reshape(EPAD // L, L)
    dst2d = jnp.concatenate([edge_index[1], pad]).reshape(EPAD // L, L)
    srcdst = jnp.stack([src2d, dst2d])
    e3d = jnp.stack([src2d, dst2d], axis=1)

    zeros128 = jnp.zeros((L, D), jnp.float32)
    ones128 = jnp.ones((L, D), jnp.float32)
    x_pad = jnp.pad(x, ((0, NP - N), (0, 0)))

    hist = _sc_hist(srcdst, zeros128, ones128)
    deg_out_col = hist[0, :, 0:1]
    deg_in_col = hist[1, :, 0:1]

    t1 = _tc_scale(x_pad, deg_out_col)
    p1 = _sc_agg(t1, e3d, zeros128)
    t2 = _tc_stage(p1, deg_in_col, deg_out_col, W1, b1.reshape(1, D),
                   relu_scale=True)
    p2 = _sc_agg(t2, e3d, zeros128)
    out = _tc_stage(p2, deg_in_col, deg_out_col, W2, b2.reshape(1, D),
                    relu_scale=False)
    return out[:N]


# speedup vs baseline: 10.8909x; 1.3328x over previous
import functools

import jax
import jax.numpy as jnp
from jax import lax
from jax.experimental import pallas as pl
from jax.experimental.pallas import tpu as pltpu
from jax.experimental.pallas import tpu_sc as plsc

N = 10000
E = 320000
D = 128

NP = 10240
TRASH = N
L = 128
CH_AGG = 80
CH_HIST = 160
EPAD = 32 * CH_AGG * L
ROWS_PER_TILE = NP // 16


def _mesh():
    return plsc.VectorSubcoreMesh(core_axis_name="c", subcore_axis_name="s")


def _sc_hist_body(srcdst, zeros1, ones1, out, idx_v, zb, ones_v, acc, sem):
    c = lax.axis_index("c")
    s = lax.axis_index("s")
    base = s * ROWS_PER_TILE

    pltpu.sync_copy(zeros1, zb)
    pltpu.sync_copy(zb, acc.at[pl.ds(base, ROWS_PER_TILE)])
    pltpu.sync_copy(ones1.at[pl.ds(0, L)], ones_v)
    pltpu.sync_copy(srcdst.at[c, pl.ds(s * CH_HIST, CH_HIST)], idx_v)
    plsc.subcore_barrier()

    K = 8

    def body(g, _):
        for k in range(K):
            pltpu.make_async_copy(
                ones_v, acc.at[idx_v.at[g * K + k]], sem).start(add=True)
        for k in range(K):
            pltpu.make_async_copy(ones_v, acc.at[idx_v.at[0]], sem).wait()
        return ()

    lax.fori_loop(0, CH_HIST // K, body, ())
    plsc.subcore_barrier()

    pltpu.sync_copy(acc.at[pl.ds(base, ROWS_PER_TILE)], zb)
    pltpu.sync_copy(zb, out.at[c, pl.ds(base, ROWS_PER_TILE)])


def _sc_hist(srcdst, zeros1, ones1):
    return pl.kernel(
        _sc_hist_body,
        out_type=jax.ShapeDtypeStruct((2, NP), jnp.float32),
        mesh=_mesh(),
        scratch_types=[
            pltpu.VMEM((CH_HIST, L), jnp.int32),
            pltpu.VMEM((ROWS_PER_TILE,), jnp.float32),
            pltpu.VMEM((L,), jnp.float32),
            pltpu.VMEM_SHARED((NP,), jnp.float32),
            pltpu.SemaphoreType.DMA,
        ],
    )(srcdst, zeros1, ones1)


def _sc_agg_body(table, e3d, zeros128, out, ibA, ibB, rowsA, rowsB, acc,
                 semA, semB):
    c = lax.axis_index("c")
    s = lax.axis_index("s")
    b = (c * 16 + s) * CH_AGG
    base = s * ROWS_PER_TILE

    pltpu.sync_copy(zeros128, rowsA)
    for j in range(ROWS_PER_TILE // L):
        pltpu.sync_copy(rowsA, acc.at[pl.ds(base + j * L, L)])
    plsc.subcore_barrier()

    pltpu.sync_copy(e3d.at[b], ibA)
    pltpu.sync_copy(e3d.at[b + 1], ibB)
    pltpu.async_copy(table.at[ibA.at[0]], rowsA, semA)
    pltpu.async_copy(table.at[ibB.at[0]], rowsB, semB)

    def half(g, ib, rows, sem):
        pltpu.make_async_copy(table.at[ib.at[0]], rows, sem).wait()
        pltpu.sync_copy(rows, acc.at[ib.at[1]], add=True)
        pltpu.sync_copy(e3d.at[g + 2], ib)
        pltpu.async_copy(table.at[ib.at[0]], rows, sem)

    def body(g, _):
        half(b + 2 * g, ibA, rowsA, semA)
        half(b + 2 * g + 1, ibB, rowsB, semB)
        return ()

    lax.fori_loop(0, CH_AGG // 2 - 1, body, ())
    pltpu.make_async_copy(table.at[ibA.at[0]], rowsA, semA).wait()
    pltpu.sync_copy(rowsA, acc.at[ibA.at[1]], add=True)
    pltpu.make_async_copy(table.at[ibB.at[0]], rowsB, semB).wait()
    pltpu.sync_copy(rowsB, acc.at[ibB.at[1]], add=True)
    plsc.subcore_barrier()

    for j in range(ROWS_PER_TILE // L):
        pltpu.sync_copy(acc.at[pl.ds(base + j * L, L)], rowsA)
        pltpu.sync_copy(rowsA, out.at[c, pl.ds(base + j * L, L)])


def _sc_agg(table, e3d, zeros128):
    return pl.kernel(
        _sc_agg_body,
        out_type=jax.ShapeDtypeStruct((2, NP, D), jnp.float32),
        mesh=_mesh(),
        scratch_types=[
            pltpu.VMEM((2, L), jnp.int32),
            pltpu.VMEM((2, L), jnp.int32),
            pltpu.VMEM((L, D), jnp.float32),
            pltpu.VMEM((L, D), jnp.float32),
            pltpu.VMEM_SHARED((NP, D), jnp.float32),
            pltpu.SemaphoreType.DMA,
            pltpu.SemaphoreType.DMA,
        ],
    )(table, e3d, zeros128)


def _norm(deg_col):
    return lax.rsqrt(jnp.where(deg_col > 0.0, deg_col, 1.0))


def _tc_scale_body(x_ref, deg_ref, o_ref):
    o_ref[...] = x_ref[...] * _norm(deg_ref[...])


def _tc_scale(x_pad, deg_out_col):
    blk = 2048
    return pl.pallas_call(
        _tc_scale_body,
        out_shape=jax.ShapeDtypeStruct((NP, D), jnp.float32),
        grid=(NP // blk,),
        in_specs=[
            pl.BlockSpec((blk, D), lambda i: (i, 0)),
            pl.BlockSpec((blk, 1), lambda i: (i, 0)),
        ],
        out_specs=pl.BlockSpec((blk, D), lambda i: (i, 0)),
    )(x_pad, deg_out_col)


def _tc_stage_body(relu_scale, p_ref, din_ref, dout_ref, w_ref, b_ref, o_ref):
    m = (p_ref[0] + p_ref[1]) * _norm(din_ref[...])
    h = jnp.dot(m, w_ref[...], preferred_element_type=jnp.float32)
    h = h + b_ref[...]
    if relu_scale:
        h = jnp.maximum(h, 0.0) * _norm(dout_ref[...])
    o_ref[...] = h


def _tc_stage(p, deg_in_col, deg_out_col, w, b, relu_scale):
    blk = 2048
    return pl.pallas_call(
        functools.partial(_tc_stage_body, relu_scale),
        out_shape=jax.ShapeDtypeStruct((NP, D), jnp.float32),
        grid=(NP // blk,),
        in_specs=[
            pl.BlockSpec((2, blk, D), lambda i: (0, i, 0)),
            pl.BlockSpec((blk, 1), lambda i: (i, 0)),
            pl.BlockSpec((blk, 1), lambda i: (i, 0)),
            pl.BlockSpec((D, D), lambda i: (0, 0)),
            pl.BlockSpec((1, D), lambda i: (0, 0)),
        ],
        out_specs=pl.BlockSpec((blk, D), lambda i: (i, 0)),
    )(p, deg_in_col, deg_out_col, w, b)


def kernel(x, edge_index, W1, b1, W2, b2):
    pad = jnp.arange(EPAD - E, dtype=jnp.int32) % (NP - N) + TRASH
    src2d = jnp.concatenate([edge_index[0], pad]).reshape(EPAD // L, L)
    dst2d = jnp.concatenate([edge_index[1], pad]).reshape(EPAD // L, L)
    srcdst = jnp.stack([src2d, dst2d])
    e3d = jnp.stack([src2d, dst2d], axis=1)

    zeros128 = jnp.zeros((L, D), jnp.float32)
    zeros1 = jnp.zeros((ROWS_PER_TILE,), jnp.float32)
    ones1 = jnp.ones((ROWS_PER_TILE,), jnp.float32)
    x_pad = jnp.pad(x, ((0, NP - N), (0, 0)))

    hist = _sc_hist(srcdst, zeros1, ones1)
    deg_out_col = hist[0, :, None]
    deg_in_col = hist[1, :, None]

    t1 = _tc_scale(x_pad, deg_out_col)
    p1 = _sc_agg(t1, e3d, zeros128)
    t2 = _tc_stage(p1, deg_in_col, deg_out_col, W1, b1.reshape(1, D),
                   relu_scale=True)
    p2 = _sc_agg(t2, e3d, zeros128)
    out = _tc_stage(p2, deg_in_col, deg_out_col, W2, b2.reshape(1, D),
                    relu_scale=False)
    return out[:N]


# speedup vs baseline: 11.8868x; 1.0914x over previous
import functools

import jax
import jax.numpy as jnp
from jax import lax
from jax.experimental import pallas as pl
from jax.experimental.pallas import tpu as pltpu
from jax.experimental.pallas import tpu_sc as plsc

N = 10000
E = 320000
D = 128

NP = 10240
TRASH = N
L = 128
CH_AGG = 80
CH_HIST = 160
EPAD = 32 * CH_AGG * L
ROWS_PER_TILE = NP // 16


def _mesh():
    return plsc.VectorSubcoreMesh(core_axis_name="c", subcore_axis_name="s")


def _sc_hist_body(srcdst, zeros1, ones1, out, idx_v, zb, ones_v, acc, sem):
    c = lax.axis_index("c")
    s = lax.axis_index("s")
    base = s * ROWS_PER_TILE

    pltpu.sync_copy(zeros1, zb)
    pltpu.sync_copy(zb, acc.at[pl.ds(base, ROWS_PER_TILE)])
    pltpu.sync_copy(ones1.at[pl.ds(0, L)], ones_v)
    pltpu.sync_copy(srcdst.at[c, pl.ds(s * CH_HIST, CH_HIST)], idx_v)
    plsc.subcore_barrier()

    K = 8

    def body(g, _):
        for k in range(K):
            pltpu.make_async_copy(
                ones_v, acc.at[idx_v.at[g * K + k]], sem).start(add=True)
        for k in range(K):
            pltpu.make_async_copy(ones_v, acc.at[idx_v.at[0]], sem).wait()
        return ()

    lax.fori_loop(0, CH_HIST // K, body, ())
    plsc.subcore_barrier()

    pltpu.sync_copy(acc.at[pl.ds(base, ROWS_PER_TILE)], zb)
    pltpu.sync_copy(zb, out.at[c, pl.ds(base, ROWS_PER_TILE)])


def _sc_hist(srcdst, zeros1, ones1):
    return pl.kernel(
        _sc_hist_body,
        out_type=jax.ShapeDtypeStruct((2, NP), jnp.float32),
        mesh=_mesh(),
        scratch_types=[
            pltpu.VMEM((CH_HIST, L), jnp.int32),
            pltpu.VMEM((ROWS_PER_TILE,), jnp.float32),
            pltpu.VMEM((L,), jnp.float32),
            pltpu.VMEM_SHARED((NP,), jnp.float32),
            pltpu.SemaphoreType.DMA,
        ],
    )(srcdst, zeros1, ones1)


HALF = CH_AGG // 2


def _sc_agg_body(table, e3d, zeros128, out, ibuf, rowsA, rowsB, acc,
                 semA, semB):
    c = lax.axis_index("c")
    s = lax.axis_index("s")
    b = (c * 16 + s) * CH_AGG
    base = s * ROWS_PER_TILE

    pltpu.sync_copy(zeros128, rowsA)
    for j in range(ROWS_PER_TILE // L):
        pltpu.sync_copy(rowsA, acc.at[pl.ds(base + j * L, L)])
    plsc.subcore_barrier()

    def run_half(h):
        pltpu.sync_copy(e3d.at[pl.ds(b + h * HALF, HALF)], ibuf)
        pltpu.async_copy(table.at[ibuf.at[0, 0]], rowsA, semA)
        pltpu.async_copy(table.at[ibuf.at[1, 0]], rowsB, semB)

        def phase(j, rows, sem):
            pltpu.make_async_copy(table.at[ibuf.at[0, 0]], rows, sem).wait()
            pltpu.sync_copy(rows, acc.at[ibuf.at[j, 1]], add=True)
            pltpu.async_copy(table.at[ibuf.at[j + 2, 0]], rows, sem)

        def body(p, _):
            phase(2 * p, rowsA, semA)
            phase(2 * p + 1, rowsB, semB)
            return ()

        lax.fori_loop(0, HALF // 2 - 1, body, ())
        pltpu.make_async_copy(table.at[ibuf.at[0, 0]], rowsA, semA).wait()
        pltpu.sync_copy(rowsA, acc.at[ibuf.at[HALF - 2, 1]], add=True)
        pltpu.make_async_copy(table.at[ibuf.at[0, 0]], rowsB, semB).wait()
        pltpu.sync_copy(rowsB, acc.at[ibuf.at[HALF - 1, 1]], add=True)

    run_half(0)
    run_half(1)
    plsc.subcore_barrier()

    for j in range(ROWS_PER_TILE // L):
        pltpu.sync_copy(acc.at[pl.ds(base + j * L, L)], rowsA)
        pltpu.sync_copy(rowsA, out.at[c, pl.ds(base + j * L, L)])


def _sc_agg(table, e3d, zeros128):
    return pl.kernel(
        _sc_agg_body,
        out_type=jax.ShapeDtypeStruct((2, NP, D), jnp.float32),
        mesh=_mesh(),
        scratch_types=[
            pltpu.VMEM((HALF, 2, L), jnp.int32),
            pltpu.VMEM((L, D), jnp.float32),
            pltpu.VMEM((L, D), jnp.float32),
            pltpu.VMEM_SHARED((NP, D), jnp.float32),
            pltpu.SemaphoreType.DMA,
            pltpu.SemaphoreType.DMA,
        ],
    )(table, e3d, zeros128)


def _norm(deg_col):
    return lax.rsqrt(jnp.where(deg_col > 0.0, deg_col, 1.0))


def _tc_scale_body(x_ref, deg_ref, o_ref):
    o_ref[...] = x_ref[...] * _norm(deg_ref[...])


def _tc_scale(x_pad, deg_out_col):
    blk = 2048
    return pl.pallas_call(
        _tc_scale_body,
        out_shape=jax.ShapeDtypeStruct((NP, D), jnp.float32),
        grid=(NP // blk,),
        in_specs=[
            pl.BlockSpec((blk, D), lambda i: (i, 0)),
            pl.BlockSpec((blk, 1), lambda i: (i, 0)),
        ],
        out_specs=pl.BlockSpec((blk, D), lambda i: (i, 0)),
    )(x_pad, deg_out_col)


def _tc_stage_body(relu_scale, p_ref, din_ref, dout_ref, w_ref, b_ref, o_ref):
    m = (p_ref[0] + p_ref[1]) * _norm(din_ref[...])
    h = jnp.dot(m, w_ref[...], preferred_element_type=jnp.float32)
    h = h + b_ref[...]
    if relu_scale:
        h = jnp.maximum(h, 0.0) * _norm(dout_ref[...])
    o_ref[...] = h


def _tc_stage(p, deg_in_col, deg_out_col, w, b, relu_scale):
    blk = 2048
    return pl.pallas_call(
        functools.partial(_tc_stage_body, relu_scale),
        out_shape=jax.ShapeDtypeStruct((NP, D), jnp.float32),
        grid=(NP // blk,),
        in_specs=[
            pl.BlockSpec((2, blk, D), lambda i: (0, i, 0)),
            pl.BlockSpec((blk, 1), lambda i: (i, 0)),
            pl.BlockSpec((blk, 1), lambda i: (i, 0)),
            pl.BlockSpec((D, D), lambda i: (0, 0)),
            pl.BlockSpec((1, D), lambda i: (0, 0)),
        ],
        out_specs=pl.BlockSpec((blk, D), lambda i: (i, 0)),
    )(p, deg_in_col, deg_out_col, w, b)


def kernel(x, edge_index, W1, b1, W2, b2):
    pad = jnp.arange(EPAD - E, dtype=jnp.int32) % (NP - N) + TRASH
    src2d = jnp.concatenate([edge_index[0], pad]).reshape(EPAD // L, L)
    dst2d = jnp.concatenate([edge_index[1], pad]).reshape(EPAD // L, L)
    srcdst = jnp.stack([src2d, dst2d])
    e3d = jnp.stack([src2d, dst2d], axis=1)

    zeros128 = jnp.zeros((L, D), jnp.float32)
    zeros1 = jnp.zeros((ROWS_PER_TILE,), jnp.float32)
    ones1 = jnp.ones((ROWS_PER_TILE,), jnp.float32)
    x_pad = jnp.pad(x, ((0, NP - N), (0, 0)))

    hist = _sc_hist(srcdst, zeros1, ones1)
    deg_out_col = hist[0, :, None]
    deg_in_col = hist[1, :, None]

    t1 = _tc_scale(x_pad, deg_out_col)
    p1 = _sc_agg(t1, e3d, zeros128)
    t2 = _tc_stage(p1, deg_in_col, deg_out_col, W1, b1.reshape(1, D),
                   relu_scale=True)
    p2 = _sc_agg(t2, e3d, zeros128)
    out = _tc_stage(p2, deg_in_col, deg_out_col, W2, b2.reshape(1, D),
                    relu_scale=False)
    return out[:N]


# speedup vs baseline: 11.9580x; 1.0060x over previous
import functools

import jax
import jax.numpy as jnp
from jax import lax
from jax.experimental import pallas as pl
from jax.experimental.pallas import tpu as pltpu
from jax.experimental.pallas import tpu_sc as plsc

N = 10000
E = 320000
D = 128

NP = 10240
TRASH = N
L = 128
CH_AGG = 80
CH_HIST = 160
EPAD = 32 * CH_AGG * L
ROWS_PER_TILE = NP // 16


def _mesh():
    return plsc.VectorSubcoreMesh(core_axis_name="c", subcore_axis_name="s")


def _sc_hist_body(srcdst, zeros1, ones1, out, idx_v, zb, ones_v, acc, sem):
    c = lax.axis_index("c")
    s = lax.axis_index("s")
    base = s * ROWS_PER_TILE

    pltpu.sync_copy(zeros1, zb)
    pltpu.sync_copy(zb, acc.at[pl.ds(base, ROWS_PER_TILE)])
    pltpu.sync_copy(ones1.at[pl.ds(0, L)], ones_v)
    pltpu.sync_copy(srcdst.at[c, pl.ds(s * CH_HIST, CH_HIST)], idx_v)
    plsc.subcore_barrier()

    K = 8

    def body(g, _):
        for k in range(K):
            pltpu.make_async_copy(
                ones_v, acc.at[idx_v.at[g * K + k]], sem).start(add=True)
        for k in range(K):
            pltpu.make_async_copy(ones_v, acc.at[idx_v.at[0]], sem).wait()
        return ()

    lax.fori_loop(0, CH_HIST // K, body, ())
    plsc.subcore_barrier()

    pltpu.sync_copy(acc.at[pl.ds(base, ROWS_PER_TILE)], zb)
    pltpu.sync_copy(zb, out.at[c, pl.ds(base, ROWS_PER_TILE)])


def _sc_hist(srcdst, zeros1, ones1):
    return pl.kernel(
        _sc_hist_body,
        out_type=jax.ShapeDtypeStruct((2, NP), jnp.float32),
        mesh=_mesh(),
        scratch_types=[
            pltpu.VMEM((CH_HIST, L), jnp.int32),
            pltpu.VMEM((ROWS_PER_TILE,), jnp.float32),
            pltpu.VMEM((L,), jnp.float32),
            pltpu.VMEM_SHARED((NP,), jnp.float32),
            pltpu.SemaphoreType.DMA,
        ],
    )(srcdst, zeros1, ones1)


HALF = CH_AGG // 2


def _sc_agg_body(table, e3d, zeros128, out, ibuf, rowsA, rowsB, acc,
                 semA, semB):
    c = lax.axis_index("c")
    s = lax.axis_index("s")
    b = (c * 16 + s) * CH_AGG
    base = s * ROWS_PER_TILE

    pltpu.sync_copy(zeros128, rowsA)
    for j in range(ROWS_PER_TILE // L):
        pltpu.make_async_copy(
            rowsA, acc.at[pl.ds(base + j * L, L)], semA).start()
    for j in range(ROWS_PER_TILE // L):
        pltpu.make_async_copy(
            rowsA, acc.at[pl.ds(base + j * L, L)], semA).wait()
    plsc.subcore_barrier()

    def run_half(h):
        pltpu.sync_copy(e3d.at[pl.ds(b + h * HALF, HALF)], ibuf)
        pltpu.async_copy(table.at[ibuf.at[0, 0]], rowsA, semA)
        pltpu.async_copy(table.at[ibuf.at[1, 0]], rowsB, semB)

        def phase(j, rows, sem):
            pltpu.make_async_copy(table.at[ibuf.at[0, 0]], rows, sem).wait()
            pltpu.sync_copy(rows, acc.at[ibuf.at[j, 1]], add=True)
            pltpu.async_copy(table.at[ibuf.at[j + 2, 0]], rows, sem)

        def body(p, _):
            phase(2 * p, rowsA, semA)
            phase(2 * p + 1, rowsB, semB)
            return ()

        lax.fori_loop(0, HALF // 2 - 1, body, ())
        pltpu.make_async_copy(table.at[ibuf.at[0, 0]], rowsA, semA).wait()
        pltpu.sync_copy(rowsA, acc.at[ibuf.at[HALF - 2, 1]], add=True)
        pltpu.make_async_copy(table.at[ibuf.at[0, 0]], rowsB, semB).wait()
        pltpu.sync_copy(rowsB, acc.at[ibuf.at[HALF - 1, 1]], add=True)

    run_half(0)
    run_half(1)
    plsc.subcore_barrier()

    def oslice(j):
        return out.at[c, pl.ds(base + j * L, L)]

    for j in range(ROWS_PER_TILE // L):
        buf, sem = (rowsA, semA) if j % 2 == 0 else (rowsB, semB)
        if j >= 2:
            pltpu.make_async_copy(buf, oslice(j - 2), sem).wait()
        pltpu.sync_copy(acc.at[pl.ds(base + j * L, L)], buf)
        pltpu.make_async_copy(buf, oslice(j), sem).start()
    pltpu.make_async_copy(rowsB, oslice(3), semB).wait()
    pltpu.make_async_copy(rowsA, oslice(4), semA).wait()


def _sc_agg(table, e3d, zeros128):
    return pl.kernel(
        _sc_agg_body,
        out_type=jax.ShapeDtypeStruct((2, NP, D), jnp.float32),
        mesh=_mesh(),
        scratch_types=[
            pltpu.VMEM((HALF, 2, L), jnp.int32),
            pltpu.VMEM((L, D), jnp.float32),
            pltpu.VMEM((L, D), jnp.float32),
            pltpu.VMEM_SHARED((NP, D), jnp.float32),
            pltpu.SemaphoreType.DMA,
            pltpu.SemaphoreType.DMA,
        ],
    )(table, e3d, zeros128)


def _norm(deg_col):
    return lax.rsqrt(jnp.where(deg_col > 0.0, deg_col, 1.0))


def _tc_scale_body(x_ref, deg_ref, o_ref):
    o_ref[...] = x_ref[...] * _norm(deg_ref[...])


def _tc_scale(x_pad, deg_out_col):
    blk = 2048
    return pl.pallas_call(
        _tc_scale_body,
        out_shape=jax.ShapeDtypeStruct((NP, D), jnp.float32),
        grid=(NP // blk,),
        in_specs=[
            pl.BlockSpec((blk, D), lambda i: (i, 0)),
            pl.BlockSpec((blk, 1), lambda i: (i, 0)),
        ],
        out_specs=pl.BlockSpec((blk, D), lambda i: (i, 0)),
    )(x_pad, deg_out_col)


def _tc_stage_body(relu_scale, p_ref, din_ref, dout_ref, w_ref, b_ref, o_ref):
    m = (p_ref[0] + p_ref[1]) * _norm(din_ref[...])
    h = jnp.dot(m, w_ref[...], preferred_element_type=jnp.float32)
    h = h + b_ref[...]
    if relu_scale:
        h = jnp.maximum(h, 0.0) * _norm(dout_ref[...])
    o_ref[...] = h


def _tc_stage(p, deg_in_col, deg_out_col, w, b, relu_scale):
    blk = 2048
    return pl.pallas_call(
        functools.partial(_tc_stage_body, relu_scale),
        out_shape=jax.ShapeDtypeStruct((NP, D), jnp.float32),
        grid=(NP // blk,),
        in_specs=[
            pl.BlockSpec((2, blk, D), lambda i: (0, i, 0)),
            pl.BlockSpec((blk, 1), lambda i: (i, 0)),
            pl.BlockSpec((blk, 1), lambda i: (i, 0)),
            pl.BlockSpec((D, D), lambda i: (0, 0)),
            pl.BlockSpec((1, D), lambda i: (0, 0)),
        ],
        out_specs=pl.BlockSpec((blk, D), lambda i: (i, 0)),
    )(p, deg_in_col, deg_out_col, w, b)


def kernel(x, edge_index, W1, b1, W2, b2):
    pad = jnp.arange(EPAD - E, dtype=jnp.int32) % (NP - N) + TRASH
    src2d = jnp.concatenate([edge_index[0], pad]).reshape(EPAD // L, L)
    dst2d = jnp.concatenate([edge_index[1], pad]).reshape(EPAD // L, L)
    srcdst = jnp.stack([src2d, dst2d])
    e3d = jnp.stack([src2d, dst2d], axis=1)

    zeros128 = jnp.zeros((L, D), jnp.float32)
    zeros1 = jnp.zeros((ROWS_PER_TILE,), jnp.float32)
    ones1 = jnp.ones((ROWS_PER_TILE,), jnp.float32)
    x_pad = jnp.pad(x, ((0, NP - N), (0, 0)))

    hist = _sc_hist(srcdst, zeros1, ones1)
    deg_out_col = hist[0, :, None]
    deg_in_col = hist[1, :, None]

    t1 = _tc_scale(x_pad, deg_out_col)
    p1 = _sc_agg(t1, e3d, zeros128)
    t2 = _tc_stage(p1, deg_in_col, deg_out_col, W1, b1.reshape(1, D),
                   relu_scale=True)
    p2 = _sc_agg(t2, e3d, zeros128)
    out = _tc_stage(p2, deg_in_col, deg_out_col, W2, b2.reshape(1, D),
                    relu_scale=False)
    return out[:N]
